# baseline (speedup 1.0000x reference)
import functools

import jax
import jax.numpy as jnp
from jax import lax
from jax.experimental import pallas as pl
from jax.experimental.pallas import tpu as pltpu
from jax.experimental.pallas import tpu_sc as plsc

BATCH = 16384
EMBED = 64
LANES = 16
NUM_CORES = 2
NUM_SUBCORES = 16
NUM_WORKERS = NUM_CORES * NUM_SUBCORES
B_PER_W = BATCH // NUM_WORKERS
HALF = B_PER_W // 2
HGROUPS = HALF // LANES
TAB2_ROWS = 500000


def _sc_body(uids_hbm, iids_hbm, utab_hbm, itab_hbm, out_hbm,
             uids_v, iids_v, uidx_v, iidx_v,
             udat_v, idat_v, out_v, sem_u, sem_i):
    wid = lax.axis_index("s") * NUM_CORES + lax.axis_index("c")
    base = wid * B_PER_W

    pltpu.sync_copy(uids_hbm.at[pl.ds(base, B_PER_W)], uids_v)
    pltpu.sync_copy(iids_hbm.at[pl.ds(base, B_PER_W)], iids_v)

    def build(g, carry):
        row = g // 8
        col = (g % 8) * LANES
        uidx_v[row, pl.ds(col, LANES)] = uids_v[pl.ds(g * LANES, LANES)] >> 1
        iidx_v[row, pl.ds(col, LANES)] = iids_v[pl.ds(g * LANES, LANES)] >> 1
        return carry

    lax.fori_loop(0, B_PER_W // LANES, build, 0)

    lane = lax.iota(jnp.int32, LANES)

    for h in range(2):
        cps = []
        for j in range(2):
            cps.append(pltpu.async_copy(
                utab_hbm.at[uidx_v.at[h * 2 + j]],
                udat_v.at[pl.ds(j * 128, 128), :], sem_u))
            cps.append(pltpu.async_copy(
                itab_hbm.at[iidx_v.at[h * 2 + j]],
                idat_v.at[pl.ds(j * 128, 128), :], sem_i))
        for cp in cps:
            cp.wait()

        def g_body(g, carry):
            tot = jnp.zeros((LANES,), jnp.float32)
            b0 = h * HALF + g * LANES
            offu_vec = (uids_v[pl.ds(b0, LANES)] & 1) * EMBED
            offi_vec = (iids_v[pl.ds(b0, LANES)] & 1) * EMBED
            for jj in range(LANES):
                r = g * LANES + jj
                off_u = offu_vec[jj]
                off_i = offi_vec[jj]
                acc = (udat_v[r, pl.ds(off_u, 16)]
                       * idat_v[r, pl.ds(off_i, 16)])
                for k in range(1, 4):
                    acc = acc + (udat_v[r, pl.ds(off_u + k * 16, 16)]
                                 * idat_v[r, pl.ds(off_i + k * 16, 16)])
                tot = jnp.where(lane == jj, jnp.sum(acc), tot)
            out_v[pl.ds(h * HALF + g * LANES, LANES)] = tot
            return carry

        lax.fori_loop(0, HGROUPS, g_body, 0)

    pltpu.sync_copy(out_v, out_hbm.at[pl.ds(base, B_PER_W)])


def kernel(user_ids, item_ids, user_table, item_table):
    mesh = plsc.VectorSubcoreMesh(core_axis_name="c", subcore_axis_name="s")
    run = functools.partial(
        pl.kernel,
        mesh=mesh,
        compiler_params=pltpu.CompilerParams(
            needs_layout_passes=False, use_tc_tiling_on_sc=True),
        out_type=jax.ShapeDtypeStruct((BATCH,), jnp.float32),
        scratch_types=[
            pltpu.VMEM((B_PER_W,), jnp.int32),
            pltpu.VMEM((B_PER_W,), jnp.int32),
            pltpu.VMEM((4, 128), jnp.int32),
            pltpu.VMEM((4, 128), jnp.int32),
            pltpu.VMEM((HALF, 128), jnp.float32),
            pltpu.VMEM((HALF, 128), jnp.float32),
            pltpu.VMEM((B_PER_W,), jnp.float32),
            pltpu.SemaphoreType.DMA,
            pltpu.SemaphoreType.DMA,
        ],
    )(_sc_body)
    return run(user_ids.astype(jnp.int32), item_ids.astype(jnp.int32),
               user_table.reshape(TAB2_ROWS, 128),
               item_table.reshape(TAB2_ROWS, 128))


# speedup vs baseline: 1.4746x; 1.4746x over previous
import functools

import jax
import jax.numpy as jnp
from jax import lax
from jax.experimental import pallas as pl
from jax.experimental.pallas import tpu as pltpu
from jax.experimental.pallas import tpu_sc as plsc

BATCH = 16384
EMBED = 64
LANES = 16
NUM_CORES = 2
NUM_SUBCORES = 16
NUM_WORKERS = NUM_CORES * NUM_SUBCORES
B_PER_W = BATCH // NUM_WORKERS
HALF = B_PER_W // 2
HGROUPS = HALF // LANES
TAB2_ROWS = 500000


def _sc_body(uids_hbm, iids_hbm, utab_hbm, itab_hbm, out_hbm,
             uids_v, iids_v, uidx_v, iidx_v,
             udat_v, idat_v, out_v, sem_u, sem_i):
    wid = lax.axis_index("s") * NUM_CORES + lax.axis_index("c")
    base = wid * B_PER_W

    pltpu.sync_copy(uids_hbm.at[pl.ds(base, B_PER_W)], uids_v)
    pltpu.sync_copy(iids_hbm.at[pl.ds(base, B_PER_W)], iids_v)

    def build(g, carry):
        row = g // 8
        col = (g % 8) * LANES
        uv = uids_v[pl.ds(g * LANES, LANES)]
        uidx_v[row, pl.ds(col, LANES)] = ((uv >> 13) << 12) | (uv & 4095)
        iidx_v[row, pl.ds(col, LANES)] = iids_v[pl.ds(g * LANES, LANES)] >> 1
        return carry

    lax.fori_loop(0, B_PER_W // LANES, build, 0)

    lane = lax.iota(jnp.int32, LANES)

    for h in range(2):
        cps = []
        for j in range(2):
            cps.append(pltpu.async_copy(
                utab_hbm.at[uidx_v.at[h * 2 + j]],
                udat_v.at[pl.ds(j * 128, 128), :], sem_u))
            cps.append(pltpu.async_copy(
                itab_hbm.at[iidx_v.at[h * 2 + j]],
                idat_v.at[pl.ds(j * 128, 128), :], sem_i))
        for cp in cps:
            cp.wait()

        def g_body(g, carry):
            tot = jnp.zeros((LANES,), jnp.float32)
            b0 = h * HALF + g * LANES
            offu_vec = ((uids_v[pl.ds(b0, LANES)] >> 12) & 1) * EMBED
            offi_vec = (iids_v[pl.ds(b0, LANES)] & 1) * EMBED
            for jj in range(LANES):
                r = g * LANES + jj
                off_u = offu_vec[jj]
                off_i = offi_vec[jj]
                acc = (udat_v[r, pl.ds(off_u, 16)]
                       * idat_v[r, pl.ds(off_i, 16)])
                for k in range(1, 4):
                    acc = acc + (udat_v[r, pl.ds(off_u + k * 16, 16)]
                                 * idat_v[r, pl.ds(off_i + k * 16, 16)])
                tot = jnp.where(lane == jj, jnp.sum(acc), tot)
            out_v[pl.ds(h * HALF + g * LANES, LANES)] = tot
            return carry

        lax.fori_loop(0, HGROUPS, g_body, 0)

    pltpu.sync_copy(out_v, out_hbm.at[pl.ds(base, B_PER_W)])


TC_BLK = 8192
TC_HB = TC_BLK // 2
TC_GRID = (1000000 + TC_BLK - 1) // TC_BLK
TCT_ROWS = TC_GRID * TC_HB


def _tc_transpose_body(in_ref, out_ref):
    x = in_ref[...]
    out_ref[:, 0:EMBED] = jnp.transpose(x[:, 0:TC_HB], (1, 0))
    out_ref[:, EMBED:128] = jnp.transpose(x[:, TC_HB:TC_BLK], (1, 0))


def _tc_transpose(tab_t):
    return pl.pallas_call(
        _tc_transpose_body,
        grid=(TC_GRID,),
        in_specs=[pl.BlockSpec((EMBED, TC_BLK), lambda i: (0, i))],
        out_specs=pl.BlockSpec((TC_HB, 128), lambda i: (i, 0)),
        out_shape=jax.ShapeDtypeStruct((TCT_ROWS, 128), jnp.float32),
    )(tab_t)


def kernel(user_ids, item_ids, user_table, item_table):
    mesh = plsc.VectorSubcoreMesh(core_axis_name="c", subcore_axis_name="s")
    run = functools.partial(
        pl.kernel,
        mesh=mesh,
        compiler_params=pltpu.CompilerParams(
            needs_layout_passes=False, use_tc_tiling_on_sc=True),
        out_type=jax.ShapeDtypeStruct((BATCH,), jnp.float32),
        scratch_types=[
            pltpu.VMEM((B_PER_W,), jnp.int32),
            pltpu.VMEM((B_PER_W,), jnp.int32),
            pltpu.VMEM((4, 128), jnp.int32),
            pltpu.VMEM((4, 128), jnp.int32),
            pltpu.VMEM((HALF, 128), jnp.float32),
            pltpu.VMEM((HALF, 128), jnp.float32),
            pltpu.VMEM((B_PER_W,), jnp.float32),
            pltpu.SemaphoreType.DMA,
            pltpu.SemaphoreType.DMA,
        ],
    )(_sc_body)
    return run(user_ids.astype(jnp.int32), item_ids.astype(jnp.int32),
               _tc_transpose(user_table.T),
               item_table.reshape(TAB2_ROWS, 128))


# speedup vs baseline: 2.0595x; 1.3967x over previous
import functools

import jax
import jax.numpy as jnp
from jax import lax
from jax.experimental import pallas as pl
from jax.experimental.pallas import tpu as pltpu
from jax.experimental.pallas import tpu_sc as plsc

BATCH = 16384
EMBED = 64
LANES = 16
NUM_CORES = 2
NUM_SUBCORES = 16
NUM_WORKERS = NUM_CORES * NUM_SUBCORES
B_PER_W = BATCH // NUM_WORKERS
HALF = B_PER_W // 2
HGROUPS = HALF // LANES
TAB2_ROWS = 500000


def _sc_body(uids_hbm, iids_hbm, utab_hbm, itab_hbm, out_hbm,
             uids_v, iids_v, uidx_v, iidx_v,
             udat_v, idat_v, out_v, sem_u, sem_i):
    wid = lax.axis_index("s") * NUM_CORES + lax.axis_index("c")
    base = wid * B_PER_W

    pltpu.sync_copy(uids_hbm.at[pl.ds(base, B_PER_W)], uids_v)
    pltpu.sync_copy(iids_hbm.at[pl.ds(base, B_PER_W)], iids_v)

    def build(g, carry):
        row = g // 8
        col = (g % 8) * LANES
        uv = uids_v[pl.ds(g * LANES, LANES)]
        iv = iids_v[pl.ds(g * LANES, LANES)]
        uidx_v[row, pl.ds(col, LANES)] = ((uv >> 13) << 12) | (uv & 4095)
        iidx_v[row, pl.ds(col, LANES)] = ((iv >> 13) << 12) | (iv & 4095)
        return carry

    lax.fori_loop(0, B_PER_W // LANES, build, 0)

    lane = lax.iota(jnp.int32, LANES)

    for h in range(2):
        cps = []
        for j in range(2):
            cps.append(pltpu.async_copy(
                utab_hbm.at[uidx_v.at[h * 2 + j]],
                udat_v.at[pl.ds(j * 128, 128), :], sem_u))
            cps.append(pltpu.async_copy(
                itab_hbm.at[iidx_v.at[h * 2 + j]],
                idat_v.at[pl.ds(j * 128, 128), :], sem_i))
        for cp in cps:
            cp.wait()

        def g_body(g, carry):
            tot = jnp.zeros((LANES,), jnp.float32)
            b0 = h * HALF + g * LANES
            offu_vec = ((uids_v[pl.ds(b0, LANES)] >> 12) & 1) * EMBED
            offi_vec = ((iids_v[pl.ds(b0, LANES)] >> 12) & 1) * EMBED
            for jj in range(LANES):
                r = g * LANES + jj
                off_u = offu_vec[jj]
                off_i = offi_vec[jj]
                acc = (udat_v[r, pl.ds(off_u, 16)]
                       * idat_v[r, pl.ds(off_i, 16)])
                for k in range(1, 4):
                    acc = acc + (udat_v[r, pl.ds(off_u + k * 16, 16)]
                                 * idat_v[r, pl.ds(off_i + k * 16, 16)])
                tot = jnp.where(lane == jj, jnp.sum(acc), tot)
            out_v[pl.ds(h * HALF + g * LANES, LANES)] = tot
            return carry

        lax.fori_loop(0, HGROUPS, g_body, 0)

    pltpu.sync_copy(out_v, out_hbm.at[pl.ds(base, B_PER_W)])


TC_BLK = 8192
TC_HB = TC_BLK // 2
TC_GRID = (1000000 + TC_BLK - 1) // TC_BLK
TCT_ROWS = TC_GRID * TC_HB


def _tc_transpose_body(in_ref, out_ref):
    x = in_ref[...]
    out_ref[:, 0:EMBED] = jnp.transpose(x[:, 0:TC_HB], (1, 0))
    out_ref[:, EMBED:128] = jnp.transpose(x[:, TC_HB:TC_BLK], (1, 0))


def _tc_transpose(tab_t):
    return pl.pallas_call(
        _tc_transpose_body,
        grid=(TC_GRID,),
        in_specs=[pl.BlockSpec((EMBED, TC_BLK), lambda i: (0, i))],
        out_specs=pl.BlockSpec((TC_HB, 128), lambda i: (i, 0)),
        out_shape=jax.ShapeDtypeStruct((TCT_ROWS, 128), jnp.float32),
    )(tab_t)


def kernel(user_ids, item_ids, user_table, item_table):
    mesh = plsc.VectorSubcoreMesh(core_axis_name="c", subcore_axis_name="s")
    run = functools.partial(
        pl.kernel,
        mesh=mesh,
        compiler_params=pltpu.CompilerParams(
            needs_layout_passes=False, use_tc_tiling_on_sc=True),
        out_type=jax.ShapeDtypeStruct((BATCH,), jnp.float32),
        scratch_types=[
            pltpu.VMEM((B_PER_W,), jnp.int32),
            pltpu.VMEM((B_PER_W,), jnp.int32),
            pltpu.VMEM((4, 128), jnp.int32),
            pltpu.VMEM((4, 128), jnp.int32),
            pltpu.VMEM((HALF, 128), jnp.float32),
            pltpu.VMEM((HALF, 128), jnp.float32),
            pltpu.VMEM((B_PER_W,), jnp.float32),
            pltpu.SemaphoreType.DMA,
            pltpu.SemaphoreType.DMA,
        ],
    )(_sc_body)
    return run(user_ids.astype(jnp.int32), item_ids.astype(jnp.int32),
               _tc_transpose(user_table.T),
               _tc_transpose(item_table.T))


# speedup vs baseline: 2.6761x; 1.2994x over previous
import functools

import jax
import jax.numpy as jnp
from jax import lax
from jax.experimental import pallas as pl
from jax.experimental.pallas import tpu as pltpu
from jax.experimental.pallas import tpu_sc as plsc

BATCH = 16384
EMBED = 64
LANES = 16
NUM_CORES = 2
NUM_SUBCORES = 16
NUM_WORKERS = NUM_CORES * NUM_SUBCORES
B_PER_W = BATCH // NUM_WORKERS
HALF = B_PER_W // 2
HGROUPS = HALF // LANES
TAB2_ROWS = 500000


def _sc_body(uids_hbm, iids_hbm, utab_hbm, itab_hbm, out_hbm,
             uids_v, iids_v, uidx_v, iidx_v,
             udat_v, idat_v, out_v, sem_u, sem_i):
    wid = lax.axis_index("s") * NUM_CORES + lax.axis_index("c")
    base = wid * B_PER_W

    pltpu.sync_copy(uids_hbm.at[pl.ds(base, B_PER_W)], uids_v)
    pltpu.sync_copy(iids_hbm.at[pl.ds(base, B_PER_W)], iids_v)

    def build(g, carry):
        row = g // 8
        col = (g % 8) * LANES
        uv = uids_v[pl.ds(g * LANES, LANES)]
        iv = iids_v[pl.ds(g * LANES, LANES)]
        uidx_v[row, pl.ds(col, LANES)] = ((uv >> 13) << 12) | (uv & 4095)
        iidx_v[row, pl.ds(col, LANES)] = ((iv >> 13) << 12) | (iv & 4095)
        return carry

    lax.fori_loop(0, B_PER_W // LANES, build, 0)

    lane = lax.iota(jnp.int32, LANES)

    for h in range(2):
        cps = []
        for j in range(2):
            cps.append(pltpu.async_copy(
                utab_hbm.at[uidx_v.at[h * 2 + j]],
                udat_v.at[pl.ds(j * 128, 128), :], sem_u))
            cps.append(pltpu.async_copy(
                itab_hbm.at[iidx_v.at[h * 2 + j]],
                idat_v.at[pl.ds(j * 128, 128), :], sem_i))
        for cp in cps:
            cp.wait()

        def g_body(g, carry):
            tot = jnp.zeros((LANES,), jnp.float32)
            b0 = h * HALF + g * LANES
            offu_vec = ((uids_v[pl.ds(b0, LANES)] >> 12) & 1) * EMBED
            offi_vec = ((iids_v[pl.ds(b0, LANES)] >> 12) & 1) * EMBED
            for jj in range(LANES):
                r = g * LANES + jj
                off_u = offu_vec[jj]
                off_i = offi_vec[jj]
                acc = (udat_v[r, pl.ds(off_u, 16)]
                       * idat_v[r, pl.ds(off_i, 16)])
                for k in range(1, 4):
                    acc = acc + (udat_v[r, pl.ds(off_u + k * 16, 16)]
                                 * idat_v[r, pl.ds(off_i + k * 16, 16)])
                tot = jnp.where(lane == jj, jnp.sum(acc), tot)
            out_v[pl.ds(h * HALF + g * LANES, LANES)] = tot
            return carry

        lax.fori_loop(0, HGROUPS, g_body, 0)

    pltpu.sync_copy(out_v, out_hbm.at[pl.ds(base, B_PER_W)])


TC_BLK = 8192
TC_HB = TC_BLK // 2
TC_GRID = (1000000 + TC_BLK - 1) // TC_BLK
TCT_ROWS = TC_GRID * TC_HB


def _tc_transpose_body(in_ref, out_ref):
    x = in_ref[...]
    y = jnp.concatenate([x[:, 0:TC_HB], x[:, TC_HB:TC_BLK]], axis=0)
    out_ref[...] = jnp.transpose(y, (1, 0))


def _tc_transpose(tab_t):
    return pl.pallas_call(
        _tc_transpose_body,
        grid=(TC_GRID,),
        in_specs=[pl.BlockSpec((EMBED, TC_BLK), lambda i: (0, i))],
        out_specs=pl.BlockSpec((TC_HB, 128), lambda i: (i, 0)),
        out_shape=jax.ShapeDtypeStruct((TCT_ROWS, 128), jnp.float32),
    )(tab_t)


def kernel(user_ids, item_ids, user_table, item_table):
    mesh = plsc.VectorSubcoreMesh(core_axis_name="c", subcore_axis_name="s")
    run = functools.partial(
        pl.kernel,
        mesh=mesh,
        compiler_params=pltpu.CompilerParams(
            needs_layout_passes=False, use_tc_tiling_on_sc=True),
        out_type=jax.ShapeDtypeStruct((BATCH,), jnp.float32),
        scratch_types=[
            pltpu.VMEM((B_PER_W,), jnp.int32),
            pltpu.VMEM((B_PER_W,), jnp.int32),
            pltpu.VMEM((4, 128), jnp.int32),
            pltpu.VMEM((4, 128), jnp.int32),
            pltpu.VMEM((HALF, 128), jnp.float32),
            pltpu.VMEM((HALF, 128), jnp.float32),
            pltpu.VMEM((B_PER_W,), jnp.float32),
            pltpu.SemaphoreType.DMA,
            pltpu.SemaphoreType.DMA,
        ],
    )(_sc_body)
    return run(user_ids.astype(jnp.int32), item_ids.astype(jnp.int32),
               _tc_transpose(user_table.T),
               _tc_transpose(item_table.T))


# speedup vs baseline: 3.0847x; 1.1527x over previous
import functools

import jax
import jax.numpy as jnp
from jax import lax
from jax.experimental import pallas as pl
from jax.experimental.pallas import tpu as pltpu
from jax.experimental.pallas import tpu_sc as plsc

BATCH = 16384
EMBED = 64
LANES = 16
NUM_CORES = 2
NUM_SUBCORES = 16
NUM_WORKERS = NUM_CORES * NUM_SUBCORES
B_PER_W = BATCH // NUM_WORKERS
HALF = B_PER_W // 2
HGROUPS = HALF // LANES
TAB2_ROWS = 500000


def _sc_body(uids_hbm, iids_hbm, utab_hbm, itab_hbm, out_hbm,
             uids_v, iids_v, uidx_v, iidx_v,
             udat_v, idat_v, out_v, sem_u, sem_i):
    wid = lax.axis_index("s") * NUM_CORES + lax.axis_index("c")
    base = wid * B_PER_W

    pltpu.sync_copy(uids_hbm.at[pl.ds(base, B_PER_W)], uids_v)
    pltpu.sync_copy(iids_hbm.at[pl.ds(base, B_PER_W)], iids_v)

    def build(g, carry):
        row = g // 8
        col = (g % 8) * LANES
        uv = uids_v[pl.ds(g * LANES, LANES)]
        iv = iids_v[pl.ds(g * LANES, LANES)]
        uidx_v[row, pl.ds(col, LANES)] = (
            ((uv >> BLK_SH) << HB_SH) | (uv & (TC_HB - 1)))
        iidx_v[row, pl.ds(col, LANES)] = (
            ((iv >> BLK_SH) << HB_SH) | (iv & (TC_HB - 1)))
        return carry

    lax.fori_loop(0, B_PER_W // LANES, build, 0)

    lane = lax.iota(jnp.int32, LANES)

    for h in range(2):
        cps = []
        for j in range(2):
            cps.append(pltpu.async_copy(
                utab_hbm.at[uidx_v.at[h * 2 + j]],
                udat_v.at[pl.ds(j * 128, 128), :], sem_u))
            cps.append(pltpu.async_copy(
                itab_hbm.at[iidx_v.at[h * 2 + j]],
                idat_v.at[pl.ds(j * 128, 128), :], sem_i))
        for cp in cps:
            cp.wait()

        def g_body(g, carry):
            tot = jnp.zeros((LANES,), jnp.float32)
            b0 = h * HALF + g * LANES
            offu_vec = ((uids_v[pl.ds(b0, LANES)] >> HB_SH) & 1) * EMBED
            offi_vec = ((iids_v[pl.ds(b0, LANES)] >> HB_SH) & 1) * EMBED
            for jj in range(LANES):
                r = g * LANES + jj
                off_u = offu_vec[jj]
                off_i = offi_vec[jj]
                acc = (udat_v[r, pl.ds(off_u, 16)]
                       * idat_v[r, pl.ds(off_i, 16)])
                for k in range(1, 4):
                    acc = acc + (udat_v[r, pl.ds(off_u + k * 16, 16)]
                                 * idat_v[r, pl.ds(off_i + k * 16, 16)])
                tot = jnp.where(lane == jj, jnp.sum(acc), tot)
            out_v[pl.ds(h * HALF + g * LANES, LANES)] = tot
            return carry

        lax.fori_loop(0, HGROUPS, g_body, 0)

    pltpu.sync_copy(out_v, out_hbm.at[pl.ds(base, B_PER_W)])


TC_BLK = 16384
TC_HB = TC_BLK // 2
TC_GRID = (1000000 + TC_BLK - 1) // TC_BLK
TCT_ROWS = TC_GRID * TC_HB
BLK_SH = TC_BLK.bit_length() - 1
HB_SH = BLK_SH - 1


def _tc_transpose_body(in_ref, out_ref):
    x = in_ref[...]
    y = jnp.concatenate([x[:, 0:TC_HB], x[:, TC_HB:TC_BLK]], axis=0)
    out_ref[...] = jnp.transpose(y, (1, 0))


def _tc_transpose(tab_t):
    return pl.pallas_call(
        _tc_transpose_body,
        grid=(TC_GRID,),
        in_specs=[pl.BlockSpec((EMBED, TC_BLK), lambda i: (0, i))],
        out_specs=pl.BlockSpec((TC_HB, 128), lambda i: (i, 0)),
        out_shape=jax.ShapeDtypeStruct((TCT_ROWS, 128), jnp.float32),
    )(tab_t)


def kernel(user_ids, item_ids, user_table, item_table):
    mesh = plsc.VectorSubcoreMesh(core_axis_name="c", subcore_axis_name="s")
    run = functools.partial(
        pl.kernel,
        mesh=mesh,
        compiler_params=pltpu.CompilerParams(
            needs_layout_passes=False, use_tc_tiling_on_sc=True),
        out_type=jax.ShapeDtypeStruct((BATCH,), jnp.float32),
        scratch_types=[
            pltpu.VMEM((B_PER_W,), jnp.int32),
            pltpu.VMEM((B_PER_W,), jnp.int32),
            pltpu.VMEM((4, 128), jnp.int32),
            pltpu.VMEM((4, 128), jnp.int32),
            pltpu.VMEM((HALF, 128), jnp.float32),
            pltpu.VMEM((HALF, 128), jnp.float32),
            pltpu.VMEM((B_PER_W,), jnp.float32),
            pltpu.SemaphoreType.DMA,
            pltpu.SemaphoreType.DMA,
        ],
    )(_sc_body)
    return run(user_ids.astype(jnp.int32), item_ids.astype(jnp.int32),
               _tc_transpose(user_table.T),
               _tc_transpose(item_table.T))


# speedup vs baseline: 3.1690x; 1.0273x over previous
import functools

import jax
import jax.numpy as jnp
from jax import lax
from jax.experimental import pallas as pl
from jax.experimental.pallas import tpu as pltpu
from jax.experimental.pallas import tpu_sc as plsc

BATCH = 16384
EMBED = 64
LANES = 16
NUM_CORES = 2
NUM_SUBCORES = 16
NUM_WORKERS = NUM_CORES * NUM_SUBCORES
B_PER_W = BATCH // NUM_WORKERS
HALF = B_PER_W // 2
HGROUPS = HALF // LANES
TAB2_ROWS = 500000


def _sc_body(uids_hbm, iids_hbm, utab_hbm, itab_hbm, out_hbm,
             uids_v, iids_v, uidx_v, iidx_v,
             udat_v, idat_v, out_v, sem_u, sem_i):
    wid = lax.axis_index("s") * NUM_CORES + lax.axis_index("c")
    base = wid * B_PER_W

    pltpu.sync_copy(uids_hbm.at[pl.ds(base, B_PER_W)], uids_v)
    pltpu.sync_copy(iids_hbm.at[pl.ds(base, B_PER_W)], iids_v)

    def build(g, carry):
        row = g // 8
        col = (g % 8) * LANES
        uv = uids_v[pl.ds(g * LANES, LANES)]
        iv = iids_v[pl.ds(g * LANES, LANES)]
        uidx_v[row, pl.ds(col, LANES)] = (
            ((uv >> BLK_SH) << HB_SH) | (uv & (TC_HB - 1)))
        iidx_v[row, pl.ds(col, LANES)] = (
            ((iv >> BLK_SH) << HB_SH) | (iv & (TC_HB - 1)))
        return carry

    lax.fori_loop(0, B_PER_W // LANES, build, 0)

    lane = lax.iota(jnp.int32, LANES)

    for h in range(2):
        cps = []
        for j in range(2):
            cps.append(pltpu.async_copy(
                utab_hbm.at[uidx_v.at[h * 2 + j]],
                udat_v.at[pl.ds(j * 128, 128), :], sem_u))
            cps.append(pltpu.async_copy(
                itab_hbm.at[iidx_v.at[h * 2 + j]],
                idat_v.at[pl.ds(j * 128, 128), :], sem_i))
        for cp in cps:
            cp.wait()

        def g_body(g, carry):
            tot = jnp.zeros((LANES,), jnp.float32)
            b0 = h * HALF + g * LANES
            offu_vec = ((uids_v[pl.ds(b0, LANES)] >> HB_SH) & 1) * EMBED
            offi_vec = ((iids_v[pl.ds(b0, LANES)] >> HB_SH) & 1) * EMBED
            for jj in range(LANES):
                r = g * LANES + jj
                off_u = offu_vec[jj]
                off_i = offi_vec[jj]
                acc = (udat_v[r, pl.ds(off_u, 16)]
                       * idat_v[r, pl.ds(off_i, 16)])
                for k in range(1, 4):
                    acc = acc + (udat_v[r, pl.ds(off_u + k * 16, 16)]
                                 * idat_v[r, pl.ds(off_i + k * 16, 16)])
                tot = jnp.where(lane == jj, jnp.sum(acc), tot)
            out_v[pl.ds(h * HALF + g * LANES, LANES)] = tot
            return carry

        lax.fori_loop(0, HGROUPS, g_body, 0)

    pltpu.sync_copy(out_v, out_hbm.at[pl.ds(base, B_PER_W)])


TC_BLK = 32768
TC_HB = TC_BLK // 2
TC_GRID = (1000000 + TC_BLK - 1) // TC_BLK
TCT_ROWS = TC_GRID * TC_HB
BLK_SH = TC_BLK.bit_length() - 1
HB_SH = BLK_SH - 1


def _tc_transpose_body(in_ref, out_ref):
    x = in_ref[...]
    y = jnp.concatenate([x[:, 0:TC_HB], x[:, TC_HB:TC_BLK]], axis=0)
    out_ref[...] = jnp.transpose(y, (1, 0))


def _tc_transpose(tab_t):
    return pl.pallas_call(
        _tc_transpose_body,
        grid=(TC_GRID,),
        in_specs=[pl.BlockSpec((EMBED, TC_BLK), lambda i: (0, i))],
        out_specs=pl.BlockSpec((TC_HB, 128), lambda i: (i, 0)),
        out_shape=jax.ShapeDtypeStruct((TCT_ROWS, 128), jnp.float32),
    )(tab_t)


def kernel(user_ids, item_ids, user_table, item_table):
    mesh = plsc.VectorSubcoreMesh(core_axis_name="c", subcore_axis_name="s")
    run = functools.partial(
        pl.kernel,
        mesh=mesh,
        compiler_params=pltpu.CompilerParams(
            needs_layout_passes=False, use_tc_tiling_on_sc=True),
        out_type=jax.ShapeDtypeStruct((BATCH,), jnp.float32),
        scratch_types=[
            pltpu.VMEM((B_PER_W,), jnp.int32),
            pltpu.VMEM((B_PER_W,), jnp.int32),
            pltpu.VMEM((4, 128), jnp.int32),
            pltpu.VMEM((4, 128), jnp.int32),
            pltpu.VMEM((HALF, 128), jnp.float32),
            pltpu.VMEM((HALF, 128), jnp.float32),
            pltpu.VMEM((B_PER_W,), jnp.float32),
            pltpu.SemaphoreType.DMA,
            pltpu.SemaphoreType.DMA,
        ],
    )(_sc_body)
    return run(user_ids.astype(jnp.int32), item_ids.astype(jnp.int32),
               _tc_transpose(user_table.T),
               _tc_transpose(item_table.T))


# speedup vs baseline: 4.1097x; 1.2969x over previous
import functools

import jax
import jax.numpy as jnp
from jax import lax
from jax.experimental import pallas as pl
from jax.experimental.pallas import tpu as pltpu
from jax.experimental.pallas import tpu_sc as plsc

BATCH = 16384
EMBED = 64
LANES = 16
NUM_CORES = 2
NUM_SUBCORES = 16
NUM_WORKERS = NUM_CORES * NUM_SUBCORES
B_PER_W = BATCH // NUM_WORKERS
HALF = B_PER_W // 2
HGROUPS = HALF // LANES
TAB2_ROWS = 500000


def _sc_body(uids_hbm, iids_hbm, utab_hbm, itab_hbm, out_hbm,
             uids_v, iids_v, uidx_v, iidx_v,
             udat_v, idat_v, out_v, sem_u, sem_i):
    wid = lax.axis_index("s") * NUM_CORES + lax.axis_index("c")
    base = wid * B_PER_W

    pltpu.sync_copy(uids_hbm.at[pl.ds(base, B_PER_W)], uids_v)
    pltpu.sync_copy(iids_hbm.at[pl.ds(base, B_PER_W)], iids_v)

    def build(g, carry):
        row = g // 8
        col = (g % 8) * LANES
        uv = uids_v[pl.ds(g * LANES, LANES)]
        iv = iids_v[pl.ds(g * LANES, LANES)]
        uidx_v[row, pl.ds(col, LANES)] = (
            ((uv >> BLK_SH) << Q_SH) | (uv & (TC_Q - 1)))
        iidx_v[row, pl.ds(col, LANES)] = (
            ((iv >> BLK_SH) << Q_SH) | (iv & (TC_Q - 1)))
        return carry

    lax.fori_loop(0, B_PER_W // LANES, build, 0)

    lane = lax.iota(jnp.int32, LANES)

    for h in range(2):
        cps = []
        for j in range(2):
            cps.append(pltpu.async_copy(
                utab_hbm.at[uidx_v.at[h * 2 + j]],
                udat_v.at[pl.ds(j * 128, 128), :], sem_u))
            cps.append(pltpu.async_copy(
                itab_hbm.at[iidx_v.at[h * 2 + j]],
                idat_v.at[pl.ds(j * 128, 128), :], sem_i))
        for cp in cps:
            cp.wait()

        def g_body(g, carry):
            tot = jnp.zeros((LANES,), jnp.float32)
            b0 = h * HALF + g * LANES
            uvv = uids_v[pl.ds(b0, LANES)]
            ivv = iids_v[pl.ds(b0, LANES)]
            offu_vec = ((uvv >> Q_SH) & 1) * EMBED
            offi_vec = ((ivv >> Q_SH) & 1) * EMBED
            pu_vec = (uvv >> (Q_SH + 1)) & 1
            pi_vec = (ivv >> (Q_SH + 1)) & 1
            for jj in range(LANES):
                r = g * LANES + jj
                off_u = offu_vec[jj]
                off_i = offi_vec[jj]
                p_u = pu_vec[jj]
                p_i = pi_vec[jj]
                acc = jnp.zeros((LANES,), jnp.float32)
                for k in range(4):
                    uw = udat_v[r, pl.ds(off_u + k * LANES, LANES)]
                    vw = idat_v[r, pl.ds(off_i + k * LANES, LANES)]
                    ub = plsc.bitcast(uw, jnp.bfloat16)
                    vb = plsc.bitcast(vw, jnp.bfloat16)
                    ue, uo = plsc.unpack(ub, format=plsc.PackFormat.INTERLEAVED)
                    ve, vo = plsc.unpack(vb, format=plsc.PackFormat.INTERLEAVED)
                    us = jnp.where(p_u == 0, ue, uo)
                    vs = jnp.where(p_i == 0, ve, vo)
                    acc = acc + us * vs
                tot = jnp.where(lane == jj, jnp.sum(acc), tot)
            out_v[pl.ds(h * HALF + g * LANES, LANES)] = tot
            return carry

        lax.fori_loop(0, HGROUPS, g_body, 0)

    pltpu.sync_copy(out_v, out_hbm.at[pl.ds(base, B_PER_W)])


TC_BLK = 32768
TC_Q = TC_BLK // 4
TC_GRID = (1000000 + TC_BLK - 1) // TC_BLK
TCT_ROWS = TC_GRID * TC_Q
BLK_SH = TC_BLK.bit_length() - 1
Q_SH = BLK_SH - 2


def _tc_transpose_body(in_ref, out_ref):
    x = in_ref[...]
    y = jnp.concatenate(
        [x[:, q * TC_Q:(q + 1) * TC_Q] for q in range(4)], axis=0)
    t = jnp.transpose(y, (1, 0))
    b = jnp.asarray(t, jnp.bfloat16)
    u = jax.lax.bitcast_convert_type(b, jnp.uint16).astype(jnp.uint32)
    out_ref[...] = (u[:, 0:128] | (u[:, 128:256] << 16)).astype(jnp.int32)


def _tc_transpose(tab_t):
    return pl.pallas_call(
        _tc_transpose_body,
        grid=(TC_GRID,),
        in_specs=[pl.BlockSpec((EMBED, TC_BLK), lambda i: (0, i))],
        out_specs=pl.BlockSpec((TC_Q, 128), lambda i: (i, 0)),
        out_shape=jax.ShapeDtypeStruct((TCT_ROWS, 128), jnp.int32),
    )(tab_t)


def kernel(user_ids, item_ids, user_table, item_table):
    mesh = plsc.VectorSubcoreMesh(core_axis_name="c", subcore_axis_name="s")
    run = functools.partial(
        pl.kernel,
        mesh=mesh,
        compiler_params=pltpu.CompilerParams(
            needs_layout_passes=False, use_tc_tiling_on_sc=True),
        out_type=jax.ShapeDtypeStruct((BATCH,), jnp.float32),
        scratch_types=[
            pltpu.VMEM((B_PER_W,), jnp.int32),
            pltpu.VMEM((B_PER_W,), jnp.int32),
            pltpu.VMEM((4, 128), jnp.int32),
            pltpu.VMEM((4, 128), jnp.int32),
            pltpu.VMEM((HALF, 128), jnp.int32),
            pltpu.VMEM((HALF, 128), jnp.int32),
            pltpu.VMEM((B_PER_W,), jnp.float32),
            pltpu.SemaphoreType.DMA,
            pltpu.SemaphoreType.DMA,
        ],
    )(_sc_body)
    return run(user_ids.astype(jnp.int32), item_ids.astype(jnp.int32),
               _tc_transpose(user_table.T),
               _tc_transpose(item_table.T))


# speedup vs baseline: 4.1321x; 1.0055x over previous
import functools

import jax
import jax.numpy as jnp
from jax import lax
from jax.experimental import pallas as pl
from jax.experimental.pallas import tpu as pltpu
from jax.experimental.pallas import tpu_sc as plsc

BATCH = 16384
EMBED = 64
LANES = 16
NUM_CORES = 2
NUM_SUBCORES = 16
NUM_WORKERS = NUM_CORES * NUM_SUBCORES
B_PER_W = BATCH // NUM_WORKERS
HALF = B_PER_W // 2
HGROUPS = HALF // LANES
TAB2_ROWS = 500000


def _sc_body(uids_hbm, iids_hbm, utab_hbm, itab_hbm, out_hbm,
             uids_v, iids_v, uidx_v, iidx_v,
             udat_v, idat_v, out_v, sem_u, sem_i):
    wid = lax.axis_index("s") * NUM_CORES + lax.axis_index("c")
    base = wid * B_PER_W

    pltpu.sync_copy(uids_hbm.at[pl.ds(base, B_PER_W)], uids_v)
    pltpu.sync_copy(iids_hbm.at[pl.ds(base, B_PER_W)], iids_v)

    def build(g, carry):
        row = g // 8
        col = (g % 8) * LANES
        uv = uids_v[pl.ds(g * LANES, LANES)]
        iv = iids_v[pl.ds(g * LANES, LANES)]
        uidx_v[row, pl.ds(col, LANES)] = (
            ((uv >> BLK_SH) << Q_SH) | (uv & (TC_Q - 1)))
        iidx_v[row, pl.ds(col, LANES)] = (
            ((iv >> BLK_SH) << Q_SH) | (iv & (TC_Q - 1)))
        return carry

    lax.fori_loop(0, B_PER_W // LANES, build, 0)

    lane = lax.iota(jnp.int32, LANES)

    for h in range(2):
        cps = []
        for j in range(2):
            cps.append(pltpu.async_copy(
                utab_hbm.at[uidx_v.at[h * 2 + j]],
                udat_v.at[pl.ds(j * 128, 128), :], sem_u))
            cps.append(pltpu.async_copy(
                itab_hbm.at[iidx_v.at[h * 2 + j]],
                idat_v.at[pl.ds(j * 128, 128), :], sem_i))
        for cp in cps:
            cp.wait()

        def g_body(g, carry):
            tot = jnp.zeros((LANES,), jnp.float32)
            b0 = h * HALF + g * LANES
            uvv = uids_v[pl.ds(b0, LANES)]
            ivv = iids_v[pl.ds(b0, LANES)]
            offu_vec = ((uvv >> Q_SH) & 1) * EMBED
            offi_vec = ((ivv >> Q_SH) & 1) * EMBED
            pu_vec = (uvv >> (Q_SH + 1)) & 1
            pi_vec = (ivv >> (Q_SH + 1)) & 1
            for jj in range(LANES):
                r = g * LANES + jj
                off_u = offu_vec[jj]
                off_i = offi_vec[jj]
                p_u = pu_vec[jj]
                p_i = pi_vec[jj]
                acc = jnp.zeros((LANES,), jnp.float32)
                for k in range(4):
                    uw = udat_v[r, pl.ds(off_u + k * LANES, LANES)]
                    vw = idat_v[r, pl.ds(off_i + k * LANES, LANES)]
                    ub = plsc.bitcast(uw, jnp.bfloat16)
                    vb = plsc.bitcast(vw, jnp.bfloat16)
                    ue, uo = plsc.unpack(ub, format=plsc.PackFormat.INTERLEAVED)
                    ve, vo = plsc.unpack(vb, format=plsc.PackFormat.INTERLEAVED)
                    us = jnp.where(p_u == 0, ue, uo)
                    vs = jnp.where(p_i == 0, ve, vo)
                    acc = acc + us * vs
                tot = jnp.where(lane == jj, jnp.sum(acc), tot)
            out_v[pl.ds(h * HALF + g * LANES, LANES)] = tot
            return carry

        lax.fori_loop(0, HGROUPS, g_body, 0)

    pltpu.sync_copy(out_v, out_hbm.at[pl.ds(base, B_PER_W)])


TC_BLK = 65536
TC_Q = TC_BLK // 4
TC_GRID = (1000000 + TC_BLK - 1) // TC_BLK
TCT_ROWS = TC_GRID * TC_Q
BLK_SH = TC_BLK.bit_length() - 1
Q_SH = BLK_SH - 2


def _tc_transpose_body(in_ref, out_ref):
    x = in_ref[...]
    y = jnp.concatenate(
        [x[:, q * TC_Q:(q + 1) * TC_Q] for q in range(4)], axis=0)
    t = jnp.transpose(y, (1, 0))
    b = jnp.asarray(t, jnp.bfloat16)
    u = jax.lax.bitcast_convert_type(b, jnp.uint16).astype(jnp.uint32)
    out_ref[...] = (u[:, 0:128] | (u[:, 128:256] << 16)).astype(jnp.int32)


def _tc_transpose(tab_t):
    return pl.pallas_call(
        _tc_transpose_body,
        grid=(TC_GRID,),
        in_specs=[pl.BlockSpec((EMBED, TC_BLK), lambda i: (0, i))],
        out_specs=pl.BlockSpec((TC_Q, 128), lambda i: (i, 0)),
        out_shape=jax.ShapeDtypeStruct((TCT_ROWS, 128), jnp.int32),
    )(tab_t)


def kernel(user_ids, item_ids, user_table, item_table):
    mesh = plsc.VectorSubcoreMesh(core_axis_name="c", subcore_axis_name="s")
    run = functools.partial(
        pl.kernel,
        mesh=mesh,
        compiler_params=pltpu.CompilerParams(
            needs_layout_passes=False, use_tc_tiling_on_sc=True),
        out_type=jax.ShapeDtypeStruct((BATCH,), jnp.float32),
        scratch_types=[
            pltpu.VMEM((B_PER_W,), jnp.int32),
            pltpu.VMEM((B_PER_W,), jnp.int32),
            pltpu.VMEM((4, 128), jnp.int32),
            pltpu.VMEM((4, 128), jnp.int32),
            pltpu.VMEM((HALF, 128), jnp.int32),
            pltpu.VMEM((HALF, 128), jnp.int32),
            pltpu.VMEM((B_PER_W,), jnp.float32),
            pltpu.SemaphoreType.DMA,
            pltpu.SemaphoreType.DMA,
        ],
    )(_sc_body)
    return run(user_ids.astype(jnp.int32), item_ids.astype(jnp.int32),
               _tc_transpose(user_table.T),
               _tc_transpose(item_table.T))


# speedup vs baseline: 4.1756x; 1.0105x over previous
import functools

import jax
import jax.numpy as jnp
from jax import lax
from jax.experimental import pallas as pl
from jax.experimental.pallas import tpu as pltpu
from jax.experimental.pallas import tpu_sc as plsc

BATCH = 16384
EMBED = 64
LANES = 16
NUM_CORES = 2
NUM_SUBCORES = 16
NUM_WORKERS = NUM_CORES * NUM_SUBCORES
B_PER_W = BATCH // NUM_WORKERS
HALF = B_PER_W // 2
HGROUPS = HALF // LANES
TAB2_ROWS = 500000


def _sc_body(uids_hbm, iids_hbm, utab_hbm, itab_hbm, out_hbm,
             uids_v, iids_v, uidx_v, iidx_v,
             udat_v, idat_v, out_v, sem_u, sem_i):
    wid = lax.axis_index("s") * NUM_CORES + lax.axis_index("c")
    base = wid * B_PER_W

    pltpu.sync_copy(uids_hbm.at[pl.ds(base, B_PER_W)], uids_v)
    pltpu.sync_copy(iids_hbm.at[pl.ds(base, B_PER_W)], iids_v)

    def build(g, carry):
        row = g // 8
        col = (g % 8) * LANES
        uv = uids_v[pl.ds(g * LANES, LANES)]
        iv = iids_v[pl.ds(g * LANES, LANES)]
        uidx_v[row, pl.ds(col, LANES)] = (
            ((uv >> BLK_SH) << Q_SH) | (uv & (TC_Q - 1)))
        iidx_v[row, pl.ds(col, LANES)] = (
            ((iv >> BLK_SH) << Q_SH) | (iv & (TC_Q - 1)))
        return carry

    lax.fori_loop(0, B_PER_W // LANES, build, 0)

    lane = lax.iota(jnp.int32, LANES)

    def fire(q):
        buf = q % 2
        return (
            pltpu.async_copy(utab_hbm.at[uidx_v.at[q]],
                             udat_v.at[pl.ds(buf * 128, 128), :], sem_u),
            pltpu.async_copy(itab_hbm.at[iidx_v.at[q]],
                             idat_v.at[pl.ds(buf * 128, 128), :], sem_i),
        )

    cps = [fire(0)]
    for q in range(4):
        if q < 3:
            cps.append(fire(q + 1))
        for cp in cps[q]:
            cp.wait()
        row_base = (q % 2) * 128

        def g_body(g, carry, q=q, row_base=row_base):
            tot = jnp.zeros((LANES,), jnp.float32)
            b0 = q * 128 + g * LANES
            uvv = uids_v[pl.ds(b0, LANES)]
            ivv = iids_v[pl.ds(b0, LANES)]
            offu_vec = ((uvv >> Q_SH) & 1) * EMBED
            offi_vec = ((ivv >> Q_SH) & 1) * EMBED
            pu_vec = (uvv >> (Q_SH + 1)) & 1
            pi_vec = (ivv >> (Q_SH + 1)) & 1
            for jj in range(LANES):
                r = row_base + g * LANES + jj
                off_u = offu_vec[jj]
                off_i = offi_vec[jj]
                p_u = pu_vec[jj]
                p_i = pi_vec[jj]
                acc = jnp.zeros((LANES,), jnp.float32)
                for k in range(4):
                    uw = udat_v[r, pl.ds(off_u + k * LANES, LANES)]
                    vw = idat_v[r, pl.ds(off_i + k * LANES, LANES)]
                    ub = plsc.bitcast(uw, jnp.bfloat16)
                    vb = plsc.bitcast(vw, jnp.bfloat16)
                    ue, uo = plsc.unpack(ub, format=plsc.PackFormat.INTERLEAVED)
                    ve, vo = plsc.unpack(vb, format=plsc.PackFormat.INTERLEAVED)
                    us = jnp.where(p_u == 0, ue, uo)
                    vs = jnp.where(p_i == 0, ve, vo)
                    acc = acc + us * vs
                tot = jnp.where(lane == jj, jnp.sum(acc), tot)
            out_v[pl.ds(q * 128 + g * LANES, LANES)] = tot
            return carry

        lax.fori_loop(0, 128 // LANES, g_body, 0)

    pltpu.sync_copy(out_v, out_hbm.at[pl.ds(base, B_PER_W)])


TC_BLK = 65536
TC_Q = TC_BLK // 4
TC_GRID = (1000000 + TC_BLK - 1) // TC_BLK
TCT_ROWS = TC_GRID * TC_Q
BLK_SH = TC_BLK.bit_length() - 1
Q_SH = BLK_SH - 2


def _tc_transpose_body(in_ref, out_ref):
    x = in_ref[...]
    y = jnp.concatenate(
        [x[:, q * TC_Q:(q + 1) * TC_Q] for q in range(4)], axis=0)
    t = jnp.transpose(y, (1, 0))
    b = jnp.asarray(t, jnp.bfloat16)
    u = jax.lax.bitcast_convert_type(b, jnp.uint16).astype(jnp.uint32)
    out_ref[...] = (u[:, 0:128] | (u[:, 128:256] << 16)).astype(jnp.int32)


def _tc_transpose(tab_t):
    return pl.pallas_call(
        _tc_transpose_body,
        grid=(TC_GRID,),
        in_specs=[pl.BlockSpec((EMBED, TC_BLK), lambda i: (0, i))],
        out_specs=pl.BlockSpec((TC_Q, 128), lambda i: (i, 0)),
        out_shape=jax.ShapeDtypeStruct((TCT_ROWS, 128), jnp.int32),
    )(tab_t)


def kernel(user_ids, item_ids, user_table, item_table):
    mesh = plsc.VectorSubcoreMesh(core_axis_name="c", subcore_axis_name="s")
    run = functools.partial(
        pl.kernel,
        mesh=mesh,
        compiler_params=pltpu.CompilerParams(
            needs_layout_passes=False, use_tc_tiling_on_sc=True),
        out_type=jax.ShapeDtypeStruct((BATCH,), jnp.float32),
        scratch_types=[
            pltpu.VMEM((B_PER_W,), jnp.int32),
            pltpu.VMEM((B_PER_W,), jnp.int32),
            pltpu.VMEM((4, 128), jnp.int32),
            pltpu.VMEM((4, 128), jnp.int32),
            pltpu.VMEM((HALF, 128), jnp.int32),
            pltpu.VMEM((HALF, 128), jnp.int32),
            pltpu.VMEM((B_PER_W,), jnp.float32),
            pltpu.SemaphoreType.DMA,
            pltpu.SemaphoreType.DMA,
        ],
    )(_sc_body)
    return run(user_ids.astype(jnp.int32), item_ids.astype(jnp.int32),
               _tc_transpose(user_table.T),
               _tc_transpose(item_table.T))


# speedup vs baseline: 4.3183x; 1.0342x over previous
import functools

import jax
import jax.numpy as jnp
from jax import lax
from jax.experimental import pallas as pl
from jax.experimental.pallas import tpu as pltpu
from jax.experimental.pallas import tpu_sc as plsc

BATCH = 16384
EMBED = 64
LANES = 16
NUM_CORES = 2
NUM_SUBCORES = 16
NUM_WORKERS = NUM_CORES * NUM_SUBCORES
B_PER_W = BATCH // NUM_WORKERS
HALF = B_PER_W // 2
HGROUPS = HALF // LANES
TAB2_ROWS = 500000


def _sc_body(uids_hbm, iids_hbm, utab_hbm, itab_hbm, out_hbm,
             uids_v, iids_v, uidx_v, iidx_v,
             udat_v, idat_v, out_v, sem_u, sem_i):
    wid = lax.axis_index("s") * NUM_CORES + lax.axis_index("c")
    base = wid * B_PER_W

    pltpu.sync_copy(uids_hbm.at[pl.ds(base, B_PER_W)], uids_v)
    pltpu.sync_copy(iids_hbm.at[pl.ds(base, B_PER_W)], iids_v)

    def build(g, carry):
        row = g // 8
        col = (g % 8) * LANES
        uv = uids_v[pl.ds(g * LANES, LANES)]
        iv = iids_v[pl.ds(g * LANES, LANES)]
        uidx_v[row, pl.ds(col, LANES)] = (
            ((uv >> BLK_SH) << Q_SH) | (uv & (TC_Q - 1)))
        iidx_v[row, pl.ds(col, LANES)] = (
            ((iv >> BLK_SH) << Q_SH) | (iv & (TC_Q - 1)))
        return carry

    lax.fori_loop(0, B_PER_W // LANES, build, 0)

    lane = lax.iota(jnp.int32, LANES)

    def fire(q):
        buf = q % 2
        return (
            pltpu.async_copy(utab_hbm.at[uidx_v.at[q]],
                             udat_v.at[pl.ds(buf * 128, 128), :], sem_u),
            pltpu.async_copy(itab_hbm.at[iidx_v.at[q]],
                             idat_v.at[pl.ds(buf * 128, 128), :], sem_i),
        )

    cps = [fire(0)]
    for q in range(4):
        if q < 3:
            cps.append(fire(q + 1))
        for cp in cps[q]:
            cp.wait()
        row_base = (q % 2) * 128

        def g_body(g, carry, q=q, row_base=row_base):
            tot = jnp.zeros((LANES,), jnp.float32)
            b0 = q * 128 + g * LANES
            uvv = uids_v[pl.ds(b0, LANES)]
            ivv = iids_v[pl.ds(b0, LANES)]
            offu_vec = ((uvv >> Q_SH) & 1) * EMBED
            offi_vec = ((ivv >> Q_SH) & 1) * EMBED
            pu_vec = (uvv >> (Q_SH + 1)) & 1
            pi_vec = (ivv >> (Q_SH + 1)) & 1
            for jj in range(LANES):
                r = row_base + g * LANES + jj
                off_u = offu_vec[jj]
                off_i = offi_vec[jj]
                p_u = pu_vec[jj]
                p_i = pi_vec[jj]
                acc = jnp.zeros((LANES,), jnp.float32)
                for k in range(4):
                    uw = udat_v[r, pl.ds(off_u + k * LANES, LANES)]
                    vw = idat_v[r, pl.ds(off_i + k * LANES, LANES)]
                    ub = plsc.bitcast(uw, jnp.bfloat16)
                    vb = plsc.bitcast(vw, jnp.bfloat16)
                    ue, uo = plsc.unpack(ub, format=plsc.PackFormat.INTERLEAVED)
                    ve, vo = plsc.unpack(vb, format=plsc.PackFormat.INTERLEAVED)
                    us = jnp.where(p_u == 0, ue, uo)
                    vs = jnp.where(p_i == 0, ve, vo)
                    acc = acc + us * vs
                tot = jnp.where(lane == jj, jnp.sum(acc), tot)
            out_v[pl.ds(q * 128 + g * LANES, LANES)] = tot
            return carry

        lax.fori_loop(0, 128 // LANES, g_body, 0)

    pltpu.sync_copy(out_v, out_hbm.at[pl.ds(base, B_PER_W)])


TC_BLK = 32768
TC_Q = TC_BLK // 4
TC_GRID = (1000000 + TC_BLK - 1) // TC_BLK
TCT_ROWS = TC_GRID * TC_Q
BLK_SH = TC_BLK.bit_length() - 1
Q_SH = BLK_SH - 2


def _tc_transpose_body(in0_ref, in1_ref, out0_ref, out1_ref):
    for in_ref, out_ref in ((in0_ref, out0_ref), (in1_ref, out1_ref)):
        x = in_ref[...]
        y = jnp.concatenate(
            [x[:, q * TC_Q:(q + 1) * TC_Q] for q in range(4)], axis=0)
        t = jnp.transpose(y, (1, 0))
        b = jnp.asarray(t, jnp.bfloat16)
        u = jax.lax.bitcast_convert_type(b, jnp.uint16).astype(jnp.uint32)
        out_ref[...] = (u[:, 0:128] | (u[:, 128:256] << 16)).astype(jnp.int32)


def _tc_transpose(tab_t):
    ispec = pl.BlockSpec((EMBED, TC_BLK), lambda i: (0, i))
    ospec = pl.BlockSpec((TC_Q, 128), lambda i: (i, 0))
    oshape = jax.ShapeDtypeStruct((TCT_ROWS, 128), jnp.int32)
    return pl.pallas_call(
        _tc_transpose_body,
        grid=(TC_GRID,),
        in_specs=[ispec, ispec],
        out_specs=[ospec, ospec],
        out_shape=[oshape, oshape],
    )(*tab_t)


def kernel(user_ids, item_ids, user_table, item_table):
    mesh = plsc.VectorSubcoreMesh(core_axis_name="c", subcore_axis_name="s")
    run = functools.partial(
        pl.kernel,
        mesh=mesh,
        compiler_params=pltpu.CompilerParams(
            needs_layout_passes=False, use_tc_tiling_on_sc=True),
        out_type=jax.ShapeDtypeStruct((BATCH,), jnp.float32),
        scratch_types=[
            pltpu.VMEM((B_PER_W,), jnp.int32),
            pltpu.VMEM((B_PER_W,), jnp.int32),
            pltpu.VMEM((4, 128), jnp.int32),
            pltpu.VMEM((4, 128), jnp.int32),
            pltpu.VMEM((HALF, 128), jnp.int32),
            pltpu.VMEM((HALF, 128), jnp.int32),
            pltpu.VMEM((B_PER_W,), jnp.float32),
            pltpu.SemaphoreType.DMA,
            pltpu.SemaphoreType.DMA,
        ],
    )(_sc_body)
    ut2, it2 = _tc_transpose((user_table.T, item_table.T))
    return run(user_ids.astype(jnp.int32), item_ids.astype(jnp.int32),
               ut2, it2)
